# Initial kernel scaffold; baseline (speedup 1.0000x reference)
import jax
import jax.numpy as jnp
from jax.experimental import pallas as pl


def kernel(x, edge_index, W1l, b1, W1r, W2l, b2, W2r, W3l, b3, W3r, Wm1, bm1, Wm2, bm2):
    raise NotImplementedError("write your pallas kernel here")


# speedup vs baseline: 15.2351x; 15.2351x over previous
import jax
import jax.numpy as jnp
from jax import lax
from jax.experimental import pallas as pl
from jax.experimental.pallas import tpu as pltpu
from jax.experimental.pallas import tpu_sc as plsc

NC = 2
NS = 16
SUB = 128
NSUB = 8
CHUNK = SUB * NSUB
SLOP = 16


def _sc_segsum(n, c_feat, e_pad, col_split):
  nrows = n + SLOP
  nw = NS if col_split else NC * NS
  per_w = e_pad // nw
  n_chunks = per_w // CHUNK
  pairs = n_chunks // 2
  odd = n_chunks % 2
  zrows = -(-nrows // NS // 8) * 8
  zlast = nrows - (NS - 1) * zrows
  wrows = -(-n // NS // 8) * 8
  wlast = n - (NS - 1) * wrows
  assert zlast > 0 and wlast > 0 and zlast % 8 == 0 and wlast % 8 == 0

  mesh = plsc.VectorSubcoreMesh(core_axis_name="c", subcore_axis_name="s")

  def body(*refs):
    if col_split:
      (feat_lo, feat_hi, src_r, dst_r, zeros_hbm, out_hbm,
       src0, src1, dst0, dst1, msg0, msg1, acc, sem0, sem1) = refs
      feats = (feat_lo, feat_hi)
    else:
      (feat, src_r, dst_r, zeros_hbm, out_hbm,
       src0, src1, dst0, dst1, msg0, msg1, acc, sem0, sem1) = refs
      feats = (feat,)
    srcb = (src0, src1)
    dstb = (dst0, dst1)
    msgb = (msg0, msg1)
    semb = (sem0, sem1)
    c = lax.axis_index("c")
    s = lax.axis_index("s")
    wid = s if col_split else s * NC + c

    z0 = pl.multiple_of(s * zrows, 8)
    @pl.when(s < NS - 1)
    def _():
      pltpu.sync_copy(zeros_hbm.at[pl.ds(z0, zrows)],
                      acc.at[pl.ds(z0, zrows)])
    @pl.when(s == NS - 1)
    def _():
      zl = (NS - 1) * zrows
      pltpu.sync_copy(zeros_hbm.at[pl.ds(zl, zlast)],
                      acc.at[pl.ds(zl, zlast)])
    plsc.subcore_barrier()

    e0 = wid * per_w

    def start(i, b):
      row0 = pl.multiple_of((e0 + i * CHUNK) // SUB, 8)
      pltpu.sync_copy(src_r.at[pl.ds(row0, NSUB)], srcb[b])
      pltpu.sync_copy(dst_r.at[pl.ds(row0, NSUB)], dstb[b])
      if col_split:
        @pl.when(c == 0)
        def _():
          for j in range(NSUB):
            pltpu.async_copy(feats[0].at[srcb[b].at[j]],
                             msgb[b].at[pl.ds(j * SUB, SUB)], semb[b])
        @pl.when(c == 1)
        def _():
          for j in range(NSUB):
            pltpu.async_copy(feats[1].at[srcb[b].at[j]],
                             msgb[b].at[pl.ds(j * SUB, SUB)], semb[b])
      else:
        for j in range(NSUB):
          pltpu.async_copy(feats[0].at[srcb[b].at[j]],
                           msgb[b].at[pl.ds(j * SUB, SUB)], semb[b])

    def finish(b):
      pltpu.make_async_copy(feats[0].at[pl.ds(0, CHUNK)], msgb[b],
                            semb[b]).wait()
      for j in range(NSUB):
        pltpu.sync_copy(msgb[b].at[pl.ds(j * SUB, SUB)],
                        acc.at[dstb[b].at[j]], add=True)

    start(0, 0)

    def pair_body(p, carry):
      i0 = 2 * p
      start(i0 + 1, 1)
      finish(0)
      if odd:
        start(i0 + 2, 0)
      else:
        @pl.when(i0 + 2 < n_chunks)
        def _():
          start(i0 + 2, 0)
      finish(1)
      return carry

    lax.fori_loop(0, pairs, pair_body, 0)
    if odd:
      finish(0)

    plsc.subcore_barrier()
    r0 = pl.multiple_of(s * wrows, 8)
    @pl.when(s < NS - 1)
    def _():
      pltpu.sync_copy(acc.at[pl.ds(r0, wrows)],
                      out_hbm.at[pl.ds(pl.multiple_of(c * n + r0, 8), wrows)])
    @pl.when(s == NS - 1)
    def _():
      rl = (NS - 1) * wrows
      pltpu.sync_copy(acc.at[pl.ds(rl, wlast)],
                      out_hbm.at[pl.ds(pl.multiple_of(c * n + rl, 8), wlast)])

  return pl.kernel(
      body,
      out_type=jax.ShapeDtypeStruct((2 * n, c_feat), jnp.float32),
      mesh=mesh,
      scratch_types=[
          pltpu.VMEM((NSUB, SUB), jnp.int32),
          pltpu.VMEM((NSUB, SUB), jnp.int32),
          pltpu.VMEM((NSUB, SUB), jnp.int32),
          pltpu.VMEM((NSUB, SUB), jnp.int32),
          pltpu.VMEM((CHUNK, c_feat), jnp.float32),
          pltpu.VMEM((CHUNK, c_feat), jnp.float32),
          pltpu.VMEM_SHARED((nrows, c_feat), jnp.float32),
          pltpu.SemaphoreType.DMA,
          pltpu.SemaphoreType.DMA,
      ],
      compiler_params=pltpu.CompilerParams(use_tc_tiling_on_sc=False),
      name=f"sc_segsum_c{c_feat}_{'col' if col_split else 'edge'}",
  )


NB = 2000


def _dot_t(a, w):
  return lax.dot_general(a, w, (((1,), (1,)), ((), ())),
                         preferred_element_type=jnp.float32)


def _tc1_body(p_ref, x_ref, w1l_ref, b1_ref, w1r_ref,
              g0_ref, g1_ref, g2_ref, g3_ref, inv_ref):
  sums = p_ref[0] + p_ref[1]
  cnt = jnp.maximum(sums[:, 4:5], 1.0)
  inv = 1.0 / cnt
  mean4 = sums[:, 0:4] * inv
  pre = _dot_t(mean4, w1l_ref[...]) + b1_ref[...][None, :] \
      + _dot_t(x_ref[...], w1r_ref[...])
  h = jnp.maximum(pre, 0.0)
  g0_ref[...] = h[:, 0:16]
  g1_ref[...] = h[:, 16:32]
  g2_ref[...] = h[:, 32:48]
  g3_ref[...] = h[:, 48:64]
  inv_ref[...] = inv


def _tc2_body(aa_ref, ab_ref, inv_ref, g0_ref, g1_ref, g2_ref, g3_ref,
              w2l_ref, b2_ref, w2r_ref, w3l_ref, h2_ref, z0_ref, z1_ref):
  m = jnp.concatenate([aa_ref[0], aa_ref[1], ab_ref[0], ab_ref[1]],
                      axis=1) * inv_ref[...]
  h1c = jnp.concatenate([g0_ref[...], g1_ref[...], g2_ref[...], g3_ref[...]],
                        axis=1)
  pre = _dot_t(m, w2l_ref[...]) + b2_ref[...][None, :] \
      + _dot_t(h1c, w2r_ref[...])
  h2 = jnp.maximum(pre, 0.0)
  h2_ref[0] = h2[:, 0:32]
  h2_ref[1] = h2[:, 32:64]
  z = _dot_t(h2, w3l_ref[...])
  z0_ref[...] = z[:, 0:16]
  z1_ref[...] = z[:, 16:32]


def _tc3_body(p_ref, inv_ref, h2_ref, x_ref, w3r_ref, b3_ref, wm1_ref,
              bm1_ref, wm2_ref, bm2_ref, out_ref):
  agg3 = jnp.concatenate([p_ref[0], p_ref[1]], axis=1)
  m3 = agg3 * inv_ref[...]
  h2c = jnp.concatenate([h2_ref[0], h2_ref[1]], axis=1)
  h3 = jnp.maximum(m3 + b3_ref[...][None, :] + _dot_t(h2c, w3r_ref[...]), 0.0)
  h4 = jnp.maximum(_dot_t(h3, wm1_ref[...]) + bm1_ref[...][None, :], 0.0)
  o = _dot_t(h4, wm2_ref[...])[:, 0:1] + bm2_ref[0]
  out_ref[...] = jnp.minimum(o, x_ref[:, 0:1])


def _full(shape):
  nd = len(shape)
  return pl.BlockSpec(shape, lambda i, _nd=nd: (0,) * _nd)


def _rows(block):
  if len(block) == 3:
    return pl.BlockSpec(block, lambda i: (0, i, 0))
  return pl.BlockSpec(block, lambda i: (i, 0))


def kernel(x, edge_index, W1l, b1, W1r, W2l, b2, W2r, W3l, b3, W3r,
           Wm1, bm1, Wm2, bm2):
  n = x.shape[0]
  e = edge_index.shape[1]
  grid = n // NB

  e_pad = ((e + NC * NS * CHUNK - 1) // (NC * NS * CHUNK)) * (NC * NS * CHUNK)
  pad = e_pad - e
  src = jnp.concatenate([edge_index[0], jnp.zeros((pad,), jnp.int32)])
  dst = jnp.concatenate([edge_index[1], jnp.full((pad,), n, jnp.int32)])
  src_r = src.reshape(e_pad // SUB, SUB)
  dst_r = dst.reshape(e_pad // SUB, SUB)
  x8 = jnp.concatenate(
      [x, jnp.ones((n, 1), jnp.float32), jnp.zeros((n, 3), jnp.float32)],
      axis=1)
  z8 = jnp.zeros((n + SLOP, 8), jnp.float32)
  z16 = jnp.zeros((n + SLOP, 16), jnp.float32)

  p1 = _sc_segsum(n, 8, e_pad, col_split=False)(x8, src_r, dst_r, z8)
  p1 = p1.reshape(2, n, 8)

  g0, g1, g2, g3, inv = pl.pallas_call(
      _tc1_body,
      grid=(grid,),
      in_specs=[_rows((2, NB, 8)), _rows((NB, 4)), _full((64, 4)),
                _full((64,)), _full((64, 4))],
      out_specs=[_rows((NB, 16))] * 4 + [_rows((NB, 1))],
      out_shape=[jax.ShapeDtypeStruct((n, 16), jnp.float32)] * 4
      + [jax.ShapeDtypeStruct((n, 1), jnp.float32)],
  )(p1, x, W1l, b1, W1r)

  seg16 = _sc_segsum(n, 16, e_pad, col_split=True)
  aa = seg16(g0, g1, src_r, dst_r, z16).reshape(2, n, 16)
  ab = seg16(g2, g3, src_r, dst_r, z16).reshape(2, n, 16)

  h2, z0, z1 = pl.pallas_call(
      _tc2_body,
      grid=(grid,),
      in_specs=[_rows((2, NB, 16)), _rows((2, NB, 16)), _rows((NB, 1))]
      + [_rows((NB, 16))] * 4
      + [_full((64, 64)), _full((64,)), _full((64, 64)), _full((32, 64))],
      out_specs=[_rows((2, NB, 32)), _rows((NB, 16)), _rows((NB, 16))],
      out_shape=[jax.ShapeDtypeStruct((2, n, 32), jnp.float32),
                 jax.ShapeDtypeStruct((n, 16), jnp.float32),
                 jax.ShapeDtypeStruct((n, 16), jnp.float32)],
  )(aa, ab, inv, g0, g1, g2, g3, W2l, b2, W2r, W3l)

  p3 = seg16(z0, z1, src_r, dst_r, z16).reshape(2, n, 16)

  out = pl.pallas_call(
      _tc3_body,
      grid=(grid,),
      in_specs=[_rows((2, NB, 16)), _rows((NB, 1)), _rows((2, NB, 32)),
                _rows((NB, 4)), _full((32, 64)), _full((32,)),
                _full((16, 32)), _full((16,)), _full((128, 16)), _full((1,))],
      out_specs=_rows((NB, 1)),
      out_shape=jax.ShapeDtypeStruct((n, 1), jnp.float32),
  )(p3, inv, h2, x, W3r, b3, Wm1, bm1,
    jnp.pad(Wm2, ((0, 127), (0, 0))), bm2)

  return out[:, 0]
